# baseline (speedup 1.0000x reference)
import functools

import jax
import jax.numpy as jnp
from jax import lax
from jax.experimental import pallas as pl
from jax.experimental.pallas import tpu as pltpu
from jax.experimental.pallas import tpu_sc as plsc

_NC = 2
_NS = 16
_NW = _NC * _NS
_LANES = 16

_CH = 4096


def _sc_interp(qin, table, params, n, h, w):
    per_w = n // _NW
    nchunk = per_w // _CH

    mesh = plsc.VectorSubcoreMesh(core_axis_name="c", subcore_axis_name="s")

    @functools.partial(
        pl.kernel,
        out_type=jax.ShapeDtypeStruct((n,), jnp.float32),
        mesh=mesh,
        scratch_types=[
            [pltpu.VMEM((2 * _CH,), jnp.float32) for _ in range(2)],
            [[pltpu.VMEM((_CH,), jnp.int32) for _ in range(4)]
             for _ in range(2)],
            [[pltpu.VMEM((_CH,), jnp.float32) for _ in range(4)]
             for _ in range(2)],
            [pltpu.VMEM((_CH,), jnp.float32) for _ in range(2)],
            [pltpu.VMEM((_CH,), jnp.float32) for _ in range(2)],
            [pltpu.VMEM((_CH,), jnp.float32) for _ in range(2)],
            pltpu.VMEM((4, _LANES), jnp.float32),
            [pltpu.SemaphoreType.DMA for _ in range(2)],
        ],
        compiler_params=pltpu.CompilerParams(needs_layout_passes=False),
    )
    def body(in_hbm, table_hbm, par_hbm, out_hbm,
             in_v, idx_v, val_v, ay_v, ax_v, out_v, par_v, sem):
        wid = lax.axis_index("s") * _NC + lax.axis_index("c")
        base_w = pl.multiple_of(wid * per_w, 8)
        pltpu.sync_copy(par_hbm, par_v)
        sy = par_v[0]
        sx = par_v[1]
        oy = par_v[2]
        ox = par_v[3]

        def prep(c, p):
            base = pl.multiple_of(base_w + c * _CH, 8)
            pltpu.sync_copy(in_hbm.at[pl.ds(2 * base, 2 * _CH)], in_v[p])

            def idx_body(i, carry2):
                sl = pl.ds(i * _LANES, _LANES)
                offy = (i >> 3) * 256 + (i & 7) * _LANES
                y = in_v[p][pl.ds(offy, _LANES)] * sy + oy
                x = in_v[p][pl.ds(offy + 128, _LANES)] * sx + ox
                fyi = jnp.minimum(y.astype(jnp.int32), h - 2)
                fxi = jnp.minimum(x.astype(jnp.int32), w - 2)
                ay_v[p][sl] = y - fyi.astype(jnp.float32)
                ax_v[p][sl] = x - fxi.astype(jnp.float32)
                lin = fyi * w + fxi
                idx_v[p][0][sl] = lin
                idx_v[p][1][sl] = lin + 1
                idx_v[p][2][sl] = lin + w
                idx_v[p][3][sl] = lin + (w + 1)
                return carry2

            lax.fori_loop(0, _CH // _LANES, idx_body, 0, unroll=2)
            for k in range(4):
                pltpu.async_copy(table_hbm.at[idx_v[p][k]], val_v[p][k],
                                 sem[p])

        def finish(c, p):
            for k in range(4):
                pltpu.make_async_copy(table_hbm.at[idx_v[p][k]],
                                      val_v[p][k], sem[p]).wait()

            def blend_body(i, carry2):
                sl = pl.ds(i * _LANES, _LANES)
                tl = val_v[p][0][sl]
                tr = val_v[p][1][sl]
                bl = val_v[p][2][sl]
                br = val_v[p][3][sl]
                ax = ax_v[p][sl]
                ay = ay_v[p][sl]
                top = tl + ax * (tr - tl)
                bot = bl + ax * (br - bl)
                out_v[p][sl] = top + ay * (bot - top)
                return carry2

            lax.fori_loop(0, _CH // _LANES, blend_body, 0, unroll=2)
            base = pl.multiple_of(base_w + c * _CH, 8)
            pltpu.sync_copy(out_v[p], out_hbm.at[pl.ds(base, _CH)])

        prep(0, 0)

        def chunk_body(c2, carry):
            e = c2 * 2
            prep(e + 1, 1)
            finish(e, 0)

            @pl.when(e + 2 < nchunk)
            def _():
                prep(e + 2, 0)

            finish(e + 1, 1)
            return carry

        lax.fori_loop(0, nchunk // 2, chunk_body, 0)

    return body(qin, table, params)


def kernel(inputs, grid, bounds):
    n = inputs.shape[0]
    h, w = grid.shape[1], grid.shape[2]
    table = grid.reshape(h * w)
    qin = inputs.reshape(n // 128, 128, 2).transpose(0, 2, 1).reshape(2 * n)
    hw1 = jnp.array([h - 1, w - 1], dtype=jnp.float32)
    scale = hw1 / (bounds[1] - bounds[0])
    off = -bounds[0] * scale
    params = jnp.broadcast_to(
        jnp.concatenate([scale, off])[:, None], (4, _LANES)
    ) + jnp.zeros((4, _LANES), jnp.float32)
    out = _sc_interp(qin, table, params, n, h, w)
    return out.reshape(n, 1)


# speedup vs baseline: 1.0357x; 1.0357x over previous
import functools

import jax
import jax.numpy as jnp
from jax import lax
from jax.experimental import pallas as pl
from jax.experimental.pallas import tpu as pltpu
from jax.experimental.pallas import tpu_sc as plsc

_NC = 2
_NS = 16
_NW = _NC * _NS
_LANES = 16

_CH = 2048


def _sc_interp(qin, table, params, n, h, w):
    per_w = n // _NW
    nchunk = per_w // _CH

    mesh = plsc.VectorSubcoreMesh(core_axis_name="c", subcore_axis_name="s")

    @functools.partial(
        pl.kernel,
        out_type=jax.ShapeDtypeStruct((n,), jnp.float32),
        mesh=mesh,
        scratch_types=[
            [pltpu.VMEM((2 * _CH,), jnp.float32) for _ in range(2)],
            [pltpu.VMEM((4 * _CH,), jnp.int32) for _ in range(2)],
            [pltpu.VMEM((4 * _CH,), jnp.float32) for _ in range(2)],
            [pltpu.VMEM((_CH,), jnp.float32) for _ in range(2)],
            [pltpu.VMEM((_CH,), jnp.float32) for _ in range(2)],
            [pltpu.VMEM((_CH,), jnp.float32) for _ in range(2)],
            pltpu.VMEM((4, _LANES), jnp.float32),
            [pltpu.SemaphoreType.DMA for _ in range(2)],
            [pltpu.SemaphoreType.DMA for _ in range(2)],
        ],
        compiler_params=pltpu.CompilerParams(needs_layout_passes=False),
    )
    def body(in_hbm, table_hbm, par_hbm, out_hbm,
             in_v, idx_v, val_v, ay_v, ax_v, out_v, par_v, sem, semi):
        wid = lax.axis_index("s") * _NC + lax.axis_index("c")
        base_w = pl.multiple_of(wid * per_w, 8)
        pltpu.sync_copy(par_hbm, par_v)
        sy = par_v[0]
        sx = par_v[1]
        oy = par_v[2]
        ox = par_v[3]

        def load_in(c, p):
            @pl.when(c < nchunk)
            def _():
                base = pl.multiple_of(base_w + c * _CH, 8)
                pltpu.async_copy(in_hbm.at[pl.ds(2 * base, 2 * _CH)],
                                 in_v[p], semi[p])

        def cg(c, p):
            base = pl.multiple_of(base_w + c * _CH, 8)
            pltpu.make_async_copy(in_hbm.at[pl.ds(2 * base, 2 * _CH)],
                                  in_v[p], semi[p]).wait()

            def idx_body(i, carry2):
                sl = pl.ds(i * _LANES, _LANES)
                offy = (i >> 3) * 256 + (i & 7) * _LANES
                y = in_v[p][pl.ds(offy, _LANES)] * sy + oy
                x = in_v[p][pl.ds(offy + 128, _LANES)] * sx + ox
                fyi = jnp.minimum(y.astype(jnp.int32), h - 2)
                fxi = jnp.minimum(x.astype(jnp.int32), w - 2)
                ay_v[p][sl] = y - fyi.astype(jnp.float32)
                ax_v[p][sl] = x - fxi.astype(jnp.float32)
                lin = fyi * w + fxi
                idx_v[p][pl.ds(i * _LANES, _LANES)] = lin
                idx_v[p][pl.ds(_CH + i * _LANES, _LANES)] = lin + 1
                idx_v[p][pl.ds(2 * _CH + i * _LANES, _LANES)] = lin + w
                idx_v[p][pl.ds(3 * _CH + i * _LANES, _LANES)] = lin + (w + 1)
                return carry2

            lax.fori_loop(0, _CH // _LANES, idx_body, 0, unroll=2)
            pltpu.async_copy(table_hbm.at[idx_v[p]], val_v[p], sem[p])

        def finish(c, p):
            pltpu.make_async_copy(table_hbm.at[idx_v[p]], val_v[p],
                                  sem[p]).wait()

            def blend_body(i, carry2):
                sl = pl.ds(i * _LANES, _LANES)
                tl = val_v[p][pl.ds(i * _LANES, _LANES)]
                tr = val_v[p][pl.ds(_CH + i * _LANES, _LANES)]
                bl = val_v[p][pl.ds(2 * _CH + i * _LANES, _LANES)]
                br = val_v[p][pl.ds(3 * _CH + i * _LANES, _LANES)]
                ax = ax_v[p][sl]
                ay = ay_v[p][sl]
                top = tl + ax * (tr - tl)
                bot = bl + ax * (br - bl)
                out_v[p][sl] = top + ay * (bot - top)
                return carry2

            lax.fori_loop(0, _CH // _LANES, blend_body, 0, unroll=2)
            base = pl.multiple_of(base_w + c * _CH, 8)
            pltpu.sync_copy(out_v[p], out_hbm.at[pl.ds(base, _CH)])

        load_in(0, 0)
        cg(0, 0)
        load_in(1, 1)

        def chunk_body(c2, carry):
            e = c2 * 2
            cg(e + 1, 1)
            load_in(e + 2, 0)
            finish(e, 0)

            @pl.when(e + 2 < nchunk)
            def _():
                cg(e + 2, 0)
                load_in(e + 3, 1)

            finish(e + 1, 1)
            return carry

        lax.fori_loop(0, nchunk // 2, chunk_body, 0)

    return body(qin, table, params)


def kernel(inputs, grid, bounds):
    n = inputs.shape[0]
    h, w = grid.shape[1], grid.shape[2]
    table = grid.reshape(h * w)
    qin = inputs.reshape(n // 128, 128, 2).transpose(0, 2, 1).reshape(2 * n)
    hw1 = jnp.array([h - 1, w - 1], dtype=jnp.float32)
    scale = hw1 / (bounds[1] - bounds[0])
    off = -bounds[0] * scale
    params = jnp.broadcast_to(
        jnp.concatenate([scale, off])[:, None], (4, _LANES)
    ) + jnp.zeros((4, _LANES), jnp.float32)
    out = _sc_interp(qin, table, params, n, h, w)
    return out.reshape(n, 1)


# speedup vs baseline: 1.0370x; 1.0013x over previous
import functools

import jax
import jax.numpy as jnp
from jax import lax
from jax.experimental import pallas as pl
from jax.experimental.pallas import tpu as pltpu
from jax.experimental.pallas import tpu_sc as plsc

_NC = 2
_NS = 16
_NW = _NC * _NS
_LANES = 16

_CH = 2048


def _sc_interp(qin, table, params, n, h, w):
    per_w = n // _NW
    nchunk = per_w // _CH

    mesh = plsc.VectorSubcoreMesh(core_axis_name="c", subcore_axis_name="s")

    @functools.partial(
        pl.kernel,
        out_type=jax.ShapeDtypeStruct((n,), jnp.float32),
        mesh=mesh,
        scratch_types=[
            [pltpu.VMEM((2 * _CH,), jnp.float32) for _ in range(2)],
            [pltpu.VMEM((4 * _CH,), jnp.int32) for _ in range(2)],
            [pltpu.VMEM((4 * _CH,), jnp.float32) for _ in range(2)],
            [pltpu.VMEM((_CH,), jnp.float32) for _ in range(2)],
            [pltpu.VMEM((_CH,), jnp.float32) for _ in range(2)],
            [pltpu.VMEM((_CH,), jnp.float32) for _ in range(2)],
            pltpu.VMEM((4, _LANES), jnp.float32),
            [pltpu.SemaphoreType.DMA for _ in range(2)],
            [pltpu.SemaphoreType.DMA for _ in range(2)],
            [pltpu.SemaphoreType.DMA for _ in range(2)],
        ],
        compiler_params=pltpu.CompilerParams(needs_layout_passes=False),
    )
    def body(in_hbm, table_hbm, par_hbm, out_hbm,
             in_v, idx_v, val_v, ay_v, ax_v, out_v, par_v, sem, semi, semo):
        wid = lax.axis_index("s") * _NC + lax.axis_index("c")
        base_w = pl.multiple_of(wid * per_w, 8)
        pltpu.sync_copy(par_hbm, par_v)
        sy = par_v[0]
        sx = par_v[1]
        oy = par_v[2]
        ox = par_v[3]

        def load_in(c, p):
            @pl.when(c < nchunk)
            def _():
                base = pl.multiple_of(base_w + c * _CH, 8)
                pltpu.async_copy(in_hbm.at[pl.ds(2 * base, 2 * _CH)],
                                 in_v[p], semi[p])

        def cg(c, p):
            base = pl.multiple_of(base_w + c * _CH, 8)
            pltpu.make_async_copy(in_hbm.at[pl.ds(2 * base, 2 * _CH)],
                                  in_v[p], semi[p]).wait()

            def idx_body(i, carry2):
                sl = pl.ds(i * _LANES, _LANES)
                offy = (i >> 3) * 256 + (i & 7) * _LANES
                y = in_v[p][pl.ds(offy, _LANES)] * sy + oy
                x = in_v[p][pl.ds(offy + 128, _LANES)] * sx + ox
                fyi = jnp.minimum(y.astype(jnp.int32), h - 2)
                fxi = jnp.minimum(x.astype(jnp.int32), w - 2)
                ay_v[p][sl] = y - fyi.astype(jnp.float32)
                ax_v[p][sl] = x - fxi.astype(jnp.float32)
                lin = fyi * w + fxi
                idx_v[p][pl.ds(i * _LANES, _LANES)] = lin
                idx_v[p][pl.ds(_CH + i * _LANES, _LANES)] = lin + 1
                idx_v[p][pl.ds(2 * _CH + i * _LANES, _LANES)] = lin + w
                idx_v[p][pl.ds(3 * _CH + i * _LANES, _LANES)] = lin + (w + 1)
                return carry2

            lax.fori_loop(0, _CH // _LANES, idx_body, 0, unroll=2)
            pltpu.async_copy(table_hbm.at[idx_v[p]], val_v[p], sem[p])

        def finish(c, p):
            pltpu.make_async_copy(table_hbm.at[idx_v[p]], val_v[p],
                                  sem[p]).wait()

            @pl.when(c >= 2)
            def _():
                pb = pl.multiple_of(base_w + (c - 2) * _CH, 8)
                pltpu.make_async_copy(out_v[p], out_hbm.at[pl.ds(pb, _CH)],
                                      semo[p]).wait()

            def blend_body(i, carry2):
                sl = pl.ds(i * _LANES, _LANES)
                tl = val_v[p][pl.ds(i * _LANES, _LANES)]
                tr = val_v[p][pl.ds(_CH + i * _LANES, _LANES)]
                bl = val_v[p][pl.ds(2 * _CH + i * _LANES, _LANES)]
                br = val_v[p][pl.ds(3 * _CH + i * _LANES, _LANES)]
                ax = ax_v[p][sl]
                ay = ay_v[p][sl]
                top = tl + ax * (tr - tl)
                bot = bl + ax * (br - bl)
                out_v[p][sl] = top + ay * (bot - top)
                return carry2

            lax.fori_loop(0, _CH // _LANES, blend_body, 0, unroll=2)
            base = pl.multiple_of(base_w + c * _CH, 8)
            pltpu.async_copy(out_v[p], out_hbm.at[pl.ds(base, _CH)], semo[p])

        load_in(0, 0)
        cg(0, 0)
        load_in(1, 1)

        def chunk_body(c2, carry):
            e = c2 * 2
            cg(e + 1, 1)
            load_in(e + 2, 0)
            finish(e, 0)

            @pl.when(e + 2 < nchunk)
            def _():
                cg(e + 2, 0)
                load_in(e + 3, 1)

            finish(e + 1, 1)
            return carry

        lax.fori_loop(0, nchunk // 2, chunk_body, 0)
        for p, c in ((0, nchunk - 2), (1, nchunk - 1)):
            pb = pl.multiple_of(base_w + c * _CH, 8)
            pltpu.make_async_copy(out_v[p], out_hbm.at[pl.ds(pb, _CH)],
                                  semo[p]).wait()

    return body(qin, table, params)


def kernel(inputs, grid, bounds):
    n = inputs.shape[0]
    h, w = grid.shape[1], grid.shape[2]
    table = grid.reshape(h * w)
    qin = inputs.reshape(n // 128, 128, 2).transpose(0, 2, 1).reshape(2 * n)
    hw1 = jnp.array([h - 1, w - 1], dtype=jnp.float32)
    scale = hw1 / (bounds[1] - bounds[0])
    off = -bounds[0] * scale
    params = jnp.broadcast_to(
        jnp.concatenate([scale, off])[:, None], (4, _LANES)
    ) + jnp.zeros((4, _LANES), jnp.float32)
    out = _sc_interp(qin, table, params, n, h, w)
    return out.reshape(n, 1)


# speedup vs baseline: 1.0423x; 1.0051x over previous
import functools

import jax
import jax.numpy as jnp
from jax import lax
from jax.experimental import pallas as pl
from jax.experimental.pallas import tpu as pltpu
from jax.experimental.pallas import tpu_sc as plsc

_NC = 2
_NS = 16
_NW = _NC * _NS
_LANES = 16

_CH = 1024


def _sc_interp(qin, table, params, n, h, w):
    per_w = n // _NW
    nchunk = per_w // _CH

    mesh = plsc.VectorSubcoreMesh(core_axis_name="c", subcore_axis_name="s")

    @functools.partial(
        pl.kernel,
        out_type=jax.ShapeDtypeStruct((n,), jnp.float32),
        mesh=mesh,
        scratch_types=[
            [pltpu.VMEM((2 * _CH,), jnp.float32) for _ in range(2)],
            [pltpu.VMEM((4 * _CH,), jnp.int32) for _ in range(2)],
            [pltpu.VMEM((4 * _CH,), jnp.float32) for _ in range(2)],
            [pltpu.VMEM((_CH,), jnp.float32) for _ in range(2)],
            [pltpu.VMEM((_CH,), jnp.float32) for _ in range(2)],
            [pltpu.VMEM((_CH,), jnp.float32) for _ in range(2)],
            pltpu.VMEM((4, _LANES), jnp.float32),
            [pltpu.SemaphoreType.DMA for _ in range(2)],
            [pltpu.SemaphoreType.DMA for _ in range(2)],
            [pltpu.SemaphoreType.DMA for _ in range(2)],
        ],
        compiler_params=pltpu.CompilerParams(needs_layout_passes=False),
    )
    def body(in_hbm, table_hbm, par_hbm, out_hbm,
             in_v, idx_v, val_v, ay_v, ax_v, out_v, par_v, sem, semi, semo):
        wid = lax.axis_index("s") * _NC + lax.axis_index("c")
        base_w = pl.multiple_of(wid * per_w, 8)
        pltpu.sync_copy(par_hbm, par_v)
        sy = par_v[0]
        sx = par_v[1]
        oy = par_v[2]
        ox = par_v[3]

        def load_in(c, p):
            @pl.when(c < nchunk)
            def _():
                base = pl.multiple_of(base_w + c * _CH, 8)
                pltpu.async_copy(in_hbm.at[pl.ds(2 * base, 2 * _CH)],
                                 in_v[p], semi[p])

        def cg(c, p):
            base = pl.multiple_of(base_w + c * _CH, 8)
            pltpu.make_async_copy(in_hbm.at[pl.ds(2 * base, 2 * _CH)],
                                  in_v[p], semi[p]).wait()

            def idx_body(i, carry2):
                sl = pl.ds(i * _LANES, _LANES)
                offy = (i >> 3) * 256 + (i & 7) * _LANES
                y = in_v[p][pl.ds(offy, _LANES)] * sy + oy
                x = in_v[p][pl.ds(offy + 128, _LANES)] * sx + ox
                fyi = jnp.minimum(y.astype(jnp.int32), h - 2)
                fxi = jnp.minimum(x.astype(jnp.int32), w - 2)
                ay_v[p][sl] = y - fyi.astype(jnp.float32)
                ax_v[p][sl] = x - fxi.astype(jnp.float32)
                lin = fyi * w + fxi
                idx_v[p][pl.ds(i * _LANES, _LANES)] = lin
                idx_v[p][pl.ds(_CH + i * _LANES, _LANES)] = lin + 1
                idx_v[p][pl.ds(2 * _CH + i * _LANES, _LANES)] = lin + w
                idx_v[p][pl.ds(3 * _CH + i * _LANES, _LANES)] = lin + (w + 1)
                return carry2

            lax.fori_loop(0, _CH // _LANES, idx_body, 0, unroll=2)
            pltpu.async_copy(table_hbm.at[idx_v[p]], val_v[p], sem[p])

        def finish(c, p):
            pltpu.make_async_copy(table_hbm.at[idx_v[p]], val_v[p],
                                  sem[p]).wait()

            @pl.when(c >= 2)
            def _():
                pb = pl.multiple_of(base_w + (c - 2) * _CH, 8)
                pltpu.make_async_copy(out_v[p], out_hbm.at[pl.ds(pb, _CH)],
                                      semo[p]).wait()

            def blend_body(i, carry2):
                sl = pl.ds(i * _LANES, _LANES)
                tl = val_v[p][pl.ds(i * _LANES, _LANES)]
                tr = val_v[p][pl.ds(_CH + i * _LANES, _LANES)]
                bl = val_v[p][pl.ds(2 * _CH + i * _LANES, _LANES)]
                br = val_v[p][pl.ds(3 * _CH + i * _LANES, _LANES)]
                ax = ax_v[p][sl]
                ay = ay_v[p][sl]
                top = tl + ax * (tr - tl)
                bot = bl + ax * (br - bl)
                out_v[p][sl] = top + ay * (bot - top)
                return carry2

            lax.fori_loop(0, _CH // _LANES, blend_body, 0, unroll=2)
            base = pl.multiple_of(base_w + c * _CH, 8)
            pltpu.async_copy(out_v[p], out_hbm.at[pl.ds(base, _CH)], semo[p])

        load_in(0, 0)
        cg(0, 0)
        load_in(1, 1)

        def chunk_body(c2, carry):
            e = c2 * 2
            cg(e + 1, 1)
            load_in(e + 2, 0)
            finish(e, 0)

            @pl.when(e + 2 < nchunk)
            def _():
                cg(e + 2, 0)
                load_in(e + 3, 1)

            finish(e + 1, 1)
            return carry

        lax.fori_loop(0, nchunk // 2, chunk_body, 0)
        for p, c in ((0, nchunk - 2), (1, nchunk - 1)):
            pb = pl.multiple_of(base_w + c * _CH, 8)
            pltpu.make_async_copy(out_v[p], out_hbm.at[pl.ds(pb, _CH)],
                                  semo[p]).wait()

    return body(qin, table, params)


def kernel(inputs, grid, bounds):
    n = inputs.shape[0]
    h, w = grid.shape[1], grid.shape[2]
    table = grid.reshape(h * w)
    qin = inputs.reshape(n // 128, 128, 2).transpose(0, 2, 1).reshape(2 * n)
    hw1 = jnp.array([h - 1, w - 1], dtype=jnp.float32)
    scale = hw1 / (bounds[1] - bounds[0])
    off = -bounds[0] * scale
    params = jnp.broadcast_to(
        jnp.concatenate([scale, off])[:, None], (4, _LANES)
    ) + jnp.zeros((4, _LANES), jnp.float32)
    out = _sc_interp(qin, table, params, n, h, w)
    return out.reshape(n, 1)


# speedup vs baseline: 1.0598x; 1.0168x over previous
import functools

import jax
import jax.numpy as jnp
from jax import lax
from jax.experimental import pallas as pl
from jax.experimental.pallas import tpu as pltpu
from jax.experimental.pallas import tpu_sc as plsc

_NC = 2
_NS = 16
_NW = _NC * _NS
_LANES = 16

_CH = 512


def _sc_interp(qin, table, params, n, h, w):
    per_w = n // _NW
    nchunk = per_w // _CH

    mesh = plsc.VectorSubcoreMesh(core_axis_name="c", subcore_axis_name="s")

    @functools.partial(
        pl.kernel,
        out_type=jax.ShapeDtypeStruct((n,), jnp.float32),
        mesh=mesh,
        scratch_types=[
            [pltpu.VMEM((2 * _CH,), jnp.float32) for _ in range(2)],
            [pltpu.VMEM((4 * _CH,), jnp.int32) for _ in range(2)],
            [pltpu.VMEM((4 * _CH,), jnp.float32) for _ in range(2)],
            [pltpu.VMEM((_CH,), jnp.float32) for _ in range(2)],
            [pltpu.VMEM((_CH,), jnp.float32) for _ in range(2)],
            [pltpu.VMEM((_CH,), jnp.float32) for _ in range(2)],
            pltpu.VMEM((4, _LANES), jnp.float32),
            [pltpu.SemaphoreType.DMA for _ in range(2)],
            [pltpu.SemaphoreType.DMA for _ in range(2)],
            [pltpu.SemaphoreType.DMA for _ in range(2)],
        ],
        compiler_params=pltpu.CompilerParams(needs_layout_passes=False),
    )
    def body(in_hbm, table_hbm, par_hbm, out_hbm,
             in_v, idx_v, val_v, ay_v, ax_v, out_v, par_v, sem, semi, semo):
        wid = lax.axis_index("s") * _NC + lax.axis_index("c")
        base_w = pl.multiple_of(wid * per_w, 8)
        pltpu.sync_copy(par_hbm, par_v)
        sy = par_v[0]
        sx = par_v[1]
        oy = par_v[2]
        ox = par_v[3]

        def load_in(c, p):
            @pl.when(c < nchunk)
            def _():
                base = pl.multiple_of(base_w + c * _CH, 8)
                pltpu.async_copy(in_hbm.at[pl.ds(2 * base, 2 * _CH)],
                                 in_v[p], semi[p])

        def cg(c, p):
            base = pl.multiple_of(base_w + c * _CH, 8)
            pltpu.make_async_copy(in_hbm.at[pl.ds(2 * base, 2 * _CH)],
                                  in_v[p], semi[p]).wait()

            def idx_body(i, carry2):
                sl = pl.ds(i * _LANES, _LANES)
                offy = (i >> 3) * 256 + (i & 7) * _LANES
                y = in_v[p][pl.ds(offy, _LANES)] * sy + oy
                x = in_v[p][pl.ds(offy + 128, _LANES)] * sx + ox
                fyi = jnp.minimum(y.astype(jnp.int32), h - 2)
                fxi = jnp.minimum(x.astype(jnp.int32), w - 2)
                ay_v[p][sl] = y - fyi.astype(jnp.float32)
                ax_v[p][sl] = x - fxi.astype(jnp.float32)
                lin = fyi * w + fxi
                idx_v[p][pl.ds(i * _LANES, _LANES)] = lin
                idx_v[p][pl.ds(_CH + i * _LANES, _LANES)] = lin + 1
                idx_v[p][pl.ds(2 * _CH + i * _LANES, _LANES)] = lin + w
                idx_v[p][pl.ds(3 * _CH + i * _LANES, _LANES)] = lin + (w + 1)
                return carry2

            lax.fori_loop(0, _CH // _LANES, idx_body, 0, unroll=2)
            pltpu.async_copy(table_hbm.at[idx_v[p]], val_v[p], sem[p])

        def finish(c, p):
            pltpu.make_async_copy(table_hbm.at[idx_v[p]], val_v[p],
                                  sem[p]).wait()

            @pl.when(c >= 2)
            def _():
                pb = pl.multiple_of(base_w + (c - 2) * _CH, 8)
                pltpu.make_async_copy(out_v[p], out_hbm.at[pl.ds(pb, _CH)],
                                      semo[p]).wait()

            def blend_body(i, carry2):
                sl = pl.ds(i * _LANES, _LANES)
                tl = val_v[p][pl.ds(i * _LANES, _LANES)]
                tr = val_v[p][pl.ds(_CH + i * _LANES, _LANES)]
                bl = val_v[p][pl.ds(2 * _CH + i * _LANES, _LANES)]
                br = val_v[p][pl.ds(3 * _CH + i * _LANES, _LANES)]
                ax = ax_v[p][sl]
                ay = ay_v[p][sl]
                top = tl + ax * (tr - tl)
                bot = bl + ax * (br - bl)
                out_v[p][sl] = top + ay * (bot - top)
                return carry2

            lax.fori_loop(0, _CH // _LANES, blend_body, 0, unroll=2)
            base = pl.multiple_of(base_w + c * _CH, 8)
            pltpu.async_copy(out_v[p], out_hbm.at[pl.ds(base, _CH)], semo[p])

        load_in(0, 0)
        cg(0, 0)
        load_in(1, 1)

        def chunk_body(c2, carry):
            e = c2 * 2
            cg(e + 1, 1)
            load_in(e + 2, 0)
            finish(e, 0)

            @pl.when(e + 2 < nchunk)
            def _():
                cg(e + 2, 0)
                load_in(e + 3, 1)

            finish(e + 1, 1)
            return carry

        lax.fori_loop(0, nchunk // 2, chunk_body, 0)
        for p, c in ((0, nchunk - 2), (1, nchunk - 1)):
            pb = pl.multiple_of(base_w + c * _CH, 8)
            pltpu.make_async_copy(out_v[p], out_hbm.at[pl.ds(pb, _CH)],
                                  semo[p]).wait()

    return body(qin, table, params)


def kernel(inputs, grid, bounds):
    n = inputs.shape[0]
    h, w = grid.shape[1], grid.shape[2]
    table = grid.reshape(h * w)
    qin = inputs.reshape(n // 128, 128, 2).transpose(0, 2, 1).reshape(2 * n)
    hw1 = jnp.array([h - 1, w - 1], dtype=jnp.float32)
    scale = hw1 / (bounds[1] - bounds[0])
    off = -bounds[0] * scale
    params = jnp.broadcast_to(
        jnp.concatenate([scale, off])[:, None], (4, _LANES)
    ) + jnp.zeros((4, _LANES), jnp.float32)
    out = _sc_interp(qin, table, params, n, h, w)
    return out.reshape(n, 1)
